# Initial kernel scaffold; baseline (speedup 1.0000x reference)
import jax
import jax.numpy as jnp
from jax.experimental import pallas as pl


def kernel(x, edge_index, W1, b1, W2, b2, W3, b3, W4, b4, W_mu, b_mu, W_logstd, b_logstd):
    raise NotImplementedError("write your pallas kernel here")


# speedup vs baseline: 11.4436x; 11.4436x over previous
import functools

import jax
import jax.numpy as jnp
from jax import lax
from jax.experimental import pallas as pl
from jax.experimental.pallas import tpu as pltpu
from jax.experimental.pallas import tpu_sc as plsc

N = 10000
E = 320000
NC, NS, L = 2, 16, 16
NW = NC * NS
EROW = 128
RW = 80
EP = NW * RW * EROW
NROWS_E = EP // EROW
ACC = 10240
ZC = ACC // NS // EROW
RB = 1000
GRID = N // RB


def _mesh():
    return plsc.VectorSubcoreMesh(core_axis_name="c", subcore_axis_name="s",
                                  num_cores=NC, num_subcores=NS)


def _fill(buf, rows, cols, value):
    v = jnp.full((L,), value, jnp.float32)
    def body(i, _):
        r = i // (cols // L)
        j = i % (cols // L)
        buf[r, pl.ds(j * L, L)] = v
        return 0
    lax.fori_loop(0, rows * (cols // L), body, 0)


@functools.lru_cache(maxsize=None)
def _make_scatter(F):
    @functools.partial(
        pl.kernel,
        out_type=jax.ShapeDtypeStruct((NC, ACC, F), jnp.float32),
        mesh=_mesh(),
        compiler_params=pltpu.CompilerParams(use_tc_tiling_on_sc=False),
        scratch_types=[
            pltpu.VMEM((RW, EROW), jnp.int32),
            pltpu.VMEM((RW, EROW), jnp.int32),
            pltpu.VMEM((EROW, F), jnp.float32),
            pltpu.VMEM((EROW, F), jnp.float32),
            pltpu.VMEM_SHARED((ACC, F), jnp.float32),
            pltpu.SemaphoreType.DMA,
            pltpu.SemaphoreType.DMA,
        ],
    )
    def scatter(g_hbm, src_hbm, dst_hbm, out_hbm, sidx, didx, bufA, bufB,
                acc, semA, semB):
        c = lax.axis_index("c")
        s = lax.axis_index("s")
        w = s * NC + c

        _fill(bufA, EROW, F, 0.0)
        def zero_chunk(k, _):
            pltpu.sync_copy(bufA, acc.at[pl.ds(s * (ACC // NS) + k * EROW, EROW)])
            return 0
        lax.fori_loop(0, ZC, zero_chunk, 0)
        plsc.subcore_barrier()

        pltpu.sync_copy(src_hbm.at[pl.ds(w * RW, RW)], sidx)
        pltpu.sync_copy(dst_hbm.at[pl.ds(w * RW, RW)], didx)

        pltpu.async_copy(g_hbm.at[sidx.at[0]], bufA, semA)
        def body(t, _):
            i = 2 * t
            pltpu.async_copy(g_hbm.at[sidx.at[i + 1]], bufB, semB)
            pltpu.make_async_copy(g_hbm.at[sidx.at[i]], bufA, semA).wait()
            pltpu.sync_copy(bufA, acc.at[didx.at[i]], add=True)
            pltpu.async_copy(g_hbm.at[sidx.at[i + 2]], bufA, semA)
            pltpu.make_async_copy(g_hbm.at[sidx.at[i + 1]], bufB, semB).wait()
            pltpu.sync_copy(bufB, acc.at[didx.at[i + 1]], add=True)
            return 0
        lax.fori_loop(0, RW // 2 - 1, body, 0)
        pltpu.async_copy(g_hbm.at[sidx.at[RW - 1]], bufB, semB)
        pltpu.make_async_copy(g_hbm.at[sidx.at[RW - 2]], bufA, semA).wait()
        pltpu.sync_copy(bufA, acc.at[didx.at[RW - 2]], add=True)
        pltpu.make_async_copy(g_hbm.at[sidx.at[RW - 1]], bufB, semB).wait()
        pltpu.sync_copy(bufB, acc.at[didx.at[RW - 1]], add=True)

        plsc.subcore_barrier()
        def out_chunk(k, _):
            off = s * (ACC // NS) + k * EROW
            pltpu.sync_copy(acc.at[pl.ds(off, EROW)], bufA)
            pltpu.sync_copy(bufA, out_hbm.at[c, pl.ds(off, EROW)])
            return 0
        lax.fori_loop(0, ZC, out_chunk, 0)

    return scatter


DEGF = 16


@functools.lru_cache(maxsize=None)
def _make_deg_scatter():
    @functools.partial(
        pl.kernel,
        out_type=jax.ShapeDtypeStruct((NC, ACC, DEGF), jnp.float32),
        mesh=_mesh(),
        compiler_params=pltpu.CompilerParams(use_tc_tiling_on_sc=False),
        scratch_types=[
            pltpu.VMEM((RW, EROW), jnp.int32),
            pltpu.VMEM((EROW, DEGF), jnp.float32),
            pltpu.VMEM((EROW, DEGF), jnp.float32),
            pltpu.VMEM_SHARED((ACC, DEGF), jnp.float32),
        ],
    )
    def deg_scatter(dst_hbm, out_hbm, didx, ones, zeros, acc):
        c = lax.axis_index("c")
        s = lax.axis_index("s")
        w = s * NC + c
        _fill(ones, EROW, DEGF, 1.0)
        _fill(zeros, EROW, DEGF, 0.0)
        def zero_chunk(k, _):
            pltpu.sync_copy(zeros,
                            acc.at[pl.ds(s * (ACC // NS) + k * EROW, EROW)])
            return 0
        lax.fori_loop(0, ZC, zero_chunk, 0)
        plsc.subcore_barrier()
        pltpu.sync_copy(dst_hbm.at[pl.ds(w * RW, RW)], didx)
        def body(i, _):
            pltpu.sync_copy(ones, acc.at[didx.at[i]], add=True)
            return 0
        lax.fori_loop(0, RW, body, 0)
        plsc.subcore_barrier()
        def out_chunk(k, _):
            off = s * (ACC // NS) + k * EROW
            pltpu.sync_copy(acc.at[pl.ds(off, EROW)], zeros)
            pltpu.sync_copy(zeros, out_hbm.at[c, pl.ds(off, EROW)])
            return 0
        lax.fori_loop(0, ZC, out_chunk, 0)

    return deg_scatter


def _row(F):
    return pl.BlockSpec((RB, F), lambda i: (i, 0))


def _part(F, core):
    return pl.BlockSpec((1, RB, F), lambda i, _c=core: (_c, i, 0))


def _full(shape):
    return pl.BlockSpec(shape, lambda i: tuple(0 for _ in shape))


def _tc(body, in_specs, out_specs, out_shape):
    if not isinstance(out_shape, (tuple, list)):
        out_specs = out_specs[0]
    return pl.pallas_call(body, grid=(GRID,), in_specs=in_specs,
                          out_specs=out_specs, out_shape=out_shape)


def _k0_body(d0, d1, x, dinv_o, g0_o):
    deg = d0[0, :, 0:1] + d1[0, :, 0:1] + 1.0
    di = lax.rsqrt(deg)
    dinv_o[...] = di
    g0_o[...] = x[...] * di


def _k0(degp, x):
    return _tc(
        _k0_body,
        [_part(DEGF, 0), _part(DEGF, 1), _row(128)],
        [_row(1), _row(128)],
        (jax.ShapeDtypeStruct((N, 1), jnp.float32),
         jax.ShapeDtypeStruct((N, 128), jnp.float32)),
    )(degp, degp, x)


def _dot(a, b):
    return jnp.dot(a, b, preferred_element_type=jnp.float32)


def _k1_body(sa0, sa1, sb0, sb1, g0, dinv, W1, b1, W2, g1_o):
    s = jnp.concatenate([sa0[0] + sa1[0], sb0[0] + sb1[0]], axis=-1)
    p0 = (s + g0[...]) * dinv[...]
    h1 = jnp.maximum(_dot(p0, W1[...]) + b1[...], 0.0)
    g1_o[...] = _dot(h1, W2[...]) * dinv[...]


def _k1(sga, sgb, g0, dinv, W1, b1, W2):
    return _tc(
        _k1_body,
        [_part(64, 0), _part(64, 1), _part(64, 0), _part(64, 1),
         _row(128), _row(1),
         _full((128, 256)), _full((1, 256)), _full((256, 128))],
        [_row(128)],
        jax.ShapeDtypeStruct((N, 128), jnp.float32),
    )(sga, sga, sgb, sgb, g0, dinv, W1, b1, W2)


def _k2_body(sa0, sa1, sb0, sb1, g, dinv, b, W, go):
    s = jnp.concatenate([sa0[0] + sa1[0], sb0[0] + sb1[0]], axis=-1)
    h = jnp.maximum((s + g[...]) * dinv[...] + b[...], 0.0)
    go[...] = _dot(h, W[...]) * dinv[...]


def _k2(sga, sgb, g1, dinv, b2, W3):
    return _tc(
        _k2_body,
        [_part(64, 0), _part(64, 1), _part(64, 0), _part(64, 1),
         _row(128), _row(1),
         _full((1, 128)), _full((128, 64))],
        [_row(64)],
        jax.ShapeDtypeStruct((N, 64), jnp.float32),
    )(sga, sga, sgb, sgb, g1, dinv, b2, W3)


def _mid_body(s0, s1, g, dinv, b, W, go):
    h = jnp.maximum((s0[0] + s1[0] + g[...]) * dinv[...] + b[...], 0.0)
    go[...] = _dot(h, W[...]) * dinv[...]


def _k3(sg, g2, dinv, b3, W4):
    return _tc(
        _mid_body,
        [_part(64, 0), _part(64, 1), _row(64), _row(1),
         _full((1, 64)), _full((64, 32))],
        [_row(32)],
        jax.ShapeDtypeStruct((N, 32), jnp.float32),
    )(sg, sg, g2, dinv, b3, W4)


def _k4_body(s0, s1, g, dinv, b, go):
    go[...] = jnp.maximum((s0[0] + s1[0] + g[...]) * dinv[...] + b[...],
                          0.0) * dinv[...]


def _k4(sg, g3, dinv, b4):
    return _tc(
        _k4_body,
        [_part(32, 0), _part(32, 1), _row(32), _row(1), _full((1, 32))],
        [_row(32)],
        jax.ShapeDtypeStruct((N, 32), jnp.float32),
    )(sg, sg, g3, dinv, b4)


def _k5_body(s0, s1, g, dinv, Wm, bm, Wl, bl, mu_o, ls_o):
    q = (s0[0] + s1[0] + g[...]) * dinv[...]
    mu_o[...] = _dot(q, Wm[...]) + bm[...]
    ls_o[...] = _dot(q, Wl[...]) + bl[...]


def _k5(sg, g4, dinv, Wm, bm, Wl, bl):
    return _tc(
        _k5_body,
        [_part(32, 0), _part(32, 1), _row(32), _row(1),
         _full((32, 16)), _full((1, 16)), _full((32, 16)), _full((1, 16))],
        [_row(16), _row(16)],
        (jax.ShapeDtypeStruct((N, 16), jnp.float32),
         jax.ShapeDtypeStruct((N, 16), jnp.float32)),
    )(sg, sg, g4, dinv, Wm, bm, Wl, bl)


def _deg_scatter(dst):
    return _make_deg_scatter()(dst)


def _scatter128(g, src, dst):
    return _make_scatter(128)(g, src, dst)


def _scatter64(g, src, dst):
    return _make_scatter(64)(g, src, dst)


def _scatter32(g, src, dst):
    return _make_scatter(32)(g, src, dst)


def kernel(x, edge_index, W1, b1, W2, b2, W3, b3, W4, b4,
           W_mu, b_mu, W_logstd, b_logstd):
    ei = edge_index.astype(jnp.int32)
    pad = EP - E
    src = jnp.concatenate(
        [ei[0], jnp.zeros((pad,), jnp.int32)]).reshape(NROWS_E, EROW)
    dst = jnp.concatenate(
        [ei[1], N + (jnp.arange(pad, dtype=jnp.int32) % (ACC - N))]
    ).reshape(NROWS_E, EROW)

    degp = _deg_scatter(dst)
    dinv, g0 = _k0(degp, x)
    g1 = _k1(_scatter64(g0[:, :64], src, dst),
             _scatter64(g0[:, 64:], src, dst),
             g0, dinv, W1, b1.reshape(1, -1), W2)
    g2 = _k2(_scatter64(g1[:, :64], src, dst),
             _scatter64(g1[:, 64:], src, dst),
             g1, dinv, b2.reshape(1, -1), W3)
    g3 = _k3(_scatter64(g2, src, dst), g2, dinv, b3.reshape(1, -1), W4)
    g4 = _k4(_scatter32(g3, src, dst), g3, dinv, b4.reshape(1, -1))
    mu, logstd = _k5(_scatter32(g4, src, dst), g4, dinv,
                     W_mu, b_mu.reshape(1, -1), W_logstd,
                     b_logstd.reshape(1, -1))
    return (mu, logstd)
